# baseline (speedup 1.0000x reference)
import functools

import jax
import jax.numpy as jnp
from jax.experimental import pallas as pl
import jax.experimental.pallas.tpu as pltpu


def _pick_block(n, target):
    best = None
    for d in range(8, min(n, target) + 1, 8):
        if n % d == 0:
            best = d
    if best is not None:
        return best
    for d in range(min(n, target), 0, -1):
        if n % d == 0:
            return d
    return n


def _gcn_kernel(x_ref, w_ref, adj_ref, out_ref, sup_ref):
    m = pl.program_id(0)

    @pl.when(m == 0)
    def _compute_support():
        sup_ref[...] = jnp.dot(
            x_ref[...], w_ref[...], preferred_element_type=jnp.float32
        ).astype(jnp.bfloat16)

    out_ref[...] = jnp.maximum(
        jnp.dot(
            adj_ref[...].astype(jnp.bfloat16),
            sup_ref[...],
            preferred_element_type=jnp.float32,
        ),
        0.0,
    )


@jax.jit
def kernel(x_features, adj, weight):
    n, in_f = x_features.shape
    out_f = weight.shape[1]

    bm = _pick_block(n, 200)
    num_m = n // bm

    return pl.pallas_call(
        _gcn_kernel,
        grid=(num_m,),
        in_specs=[
            pl.BlockSpec((n, in_f), lambda m: (0, 0)),
            pl.BlockSpec((in_f, out_f), lambda m: (0, 0)),
            pl.BlockSpec((bm, n), lambda m: (m, 0)),
        ],
        out_specs=pl.BlockSpec((bm, out_f), lambda m: (m, 0)),
        out_shape=jax.ShapeDtypeStruct((n, out_f), jnp.float32),
        scratch_shapes=[pltpu.VMEM((n, out_f), jnp.bfloat16)],
    )(x_features, weight, adj)


# speedup vs baseline: 1.0210x; 1.0210x over previous
import functools

import jax
import jax.numpy as jnp
from jax.experimental import pallas as pl
import jax.experimental.pallas.tpu as pltpu


def _pick_block(n, target):
    best = None
    for d in range(8, min(n, target) + 1, 8):
        if n % d == 0:
            best = d
    if best is not None:
        return best
    for d in range(min(n, target), 0, -1):
        if n % d == 0:
            return d
    return n


def _gcn_kernel(x_ref, w_ref, adj_ref, out_ref, sup_ref):
    m = pl.program_id(0)

    @pl.when(m == 0)
    def _compute_support():
        sup_ref[...] = jnp.dot(
            x_ref[...].astype(jnp.bfloat16),
            w_ref[...].astype(jnp.bfloat16),
            preferred_element_type=jnp.float32,
        ).astype(jnp.bfloat16)

    out_ref[...] = jnp.maximum(
        jnp.dot(
            adj_ref[...].astype(jnp.bfloat16),
            sup_ref[...],
            preferred_element_type=jnp.float32,
        ),
        0.0,
    )


@jax.jit
def kernel(x_features, adj, weight):
    n, in_f = x_features.shape
    out_f = weight.shape[1]

    bm = _pick_block(n, 400)
    num_m = n // bm

    return pl.pallas_call(
        _gcn_kernel,
        grid=(num_m,),
        in_specs=[
            pl.BlockSpec((n, in_f), lambda m: (0, 0)),
            pl.BlockSpec((in_f, out_f), lambda m: (0, 0)),
            pl.BlockSpec((bm, n), lambda m: (m, 0)),
        ],
        out_specs=pl.BlockSpec((bm, out_f), lambda m: (m, 0)),
        out_shape=jax.ShapeDtypeStruct((n, out_f), jnp.float32),
        scratch_shapes=[pltpu.VMEM((n, out_f), jnp.bfloat16)],
        compiler_params=pltpu.CompilerParams(vmem_limit_bytes=110 * 1024 * 1024),
    )(x_features, weight, adj)
